# baseline (speedup 1.0000x reference)
import functools

import jax
import jax.numpy as jnp
from jax import lax
from jax.experimental import pallas as pl
from jax.experimental.pallas import tpu as pltpu
from jax.experimental.pallas import tpu_sc as plsc

B, S = 4096, 50
NG = 32
NQ = 64
D = 128
T = B * S
NW = 32
TPW = T // NW
CH = 128
NCH = TPW // CH
L = 16


RB = 2048
G = T // RB


IR = RB // 128


def _decode_body(g_ref, q_ref, gi_ref, q0_ref, q1_ref):
    iota_g = lax.broadcasted_iota(jnp.int32, (1, NG), 1).astype(jnp.float32)
    gi_ref[...] = jnp.sum(g_ref[...] * iota_g,
                          axis=1).astype(jnp.int32).reshape(IR, 128)
    iota_q = lax.broadcasted_iota(
        jnp.int32, (1, 1, NQ), 2).astype(jnp.float32)
    qidx = jnp.sum(q_ref[...] * iota_q, axis=2).astype(jnp.int32)
    q0_ref[...] = qidx[:, 0].reshape(IR, 128)
    q1_ref[...] = qidx[:, 1].reshape(IR, 128)


_decode = pl.pallas_call(
    _decode_body,
    grid=(G,),
    in_specs=[
        pl.BlockSpec((RB, NG), lambda i: (i, 0)),
        pl.BlockSpec((RB, 2, NQ), lambda i: (i, 0, 0)),
    ],
    out_specs=[
        pl.BlockSpec((IR, 128), lambda i: (i, 0)),
        pl.BlockSpec((IR, 128), lambda i: (i, 0)),
        pl.BlockSpec((IR, 128), lambda i: (i, 0)),
    ],
    out_shape=[jax.ShapeDtypeStruct((T // 128, 128), jnp.int32)] * 3,
)


_mesh = plsc.VectorSubcoreMesh(core_axis_name="c", subcore_axis_name="s")


@functools.partial(
    pl.kernel,
    out_type=jax.ShapeDtypeStruct((T, 2 * D), jnp.float32),
    mesh=_mesh,
    scratch_types=[
        pltpu.VMEM((CH,), jnp.int32),
        pltpu.VMEM((CH,), jnp.int32),
        pltpu.VMEM((CH,), jnp.int32),
        pltpu.VMEM((CH, 2 * D), jnp.float32),
        pltpu.VMEM((CH, D), jnp.float32),
        pltpu.VMEM((CH, D), jnp.float32),
        pltpu.VMEM((CH, 2 * D), jnp.float32),
        pltpu.SemaphoreType.DMA,
    ],
)
def _sc_embed(gi_hbm, q0_hbm, q1_hbm, w_hbm, qt_hbm, out_hbm,
              gi_v, q0i_v, q1i_v, wr_v, qr0_v, qr1_v, o_v, sem):
    wid = lax.axis_index("s") * 2 + lax.axis_index("c")
    base = wid * TPW

    def chunk(ci, carry):
        t0 = base + ci * CH
        pltpu.sync_copy(gi_hbm.at[pl.ds(t0, CH)], gi_v)
        pltpu.sync_copy(q0_hbm.at[pl.ds(t0, CH)], q0i_v)
        pltpu.sync_copy(q1_hbm.at[pl.ds(t0, CH)], q1i_v)
        pltpu.async_copy(w_hbm.at[gi_v], wr_v, sem).wait()
        pltpu.async_copy(qt_hbm.at[q0i_v], qr0_v, sem).wait()
        pltpu.async_copy(qt_hbm.at[q1i_v], qr1_v, sem).wait()

        def body_s(s, c2):
            for j in range(D // L):
                o_v[s, pl.ds(j * L, L)] = (
                    wr_v[s, pl.ds(j * L, L)] + qr0_v[s, pl.ds(j * L, L)])
                o_v[s, pl.ds(D + j * L, L)] = (
                    wr_v[s, pl.ds(D + j * L, L)] + qr1_v[s, pl.ds(j * L, L)])
            return c2

        lax.fori_loop(0, CH, body_s, 0)
        pltpu.sync_copy(o_v, out_hbm.at[pl.ds(t0, CH)])
        return carry

    lax.fori_loop(0, NCH, chunk, 0)


def kernel(gates_oh, gate_qubits_oh, qubits, W_gate):
    g_flat = gates_oh.reshape(T, NG)
    q_flat = gate_qubits_oh.reshape(T, 2, NQ)
    gi, q0i, q1i = _decode(g_flat, q_flat)
    out = _sc_embed(gi.reshape(T), q0i.reshape(T), q1i.reshape(T),
                    W_gate, qubits)
    return out.reshape(B, S, 2 * D)


# speedup vs baseline: 1.4088x; 1.4088x over previous
import functools

import jax
import jax.numpy as jnp
from jax import lax
from jax.experimental import pallas as pl
from jax.experimental.pallas import tpu as pltpu
from jax.experimental.pallas import tpu_sc as plsc

B, S = 4096, 50
NG = 32
NQ = 64
D = 128
T = B * S
NW = 32
TPW = T // NW
CH = 128
NCH = TPW // CH
L = 16


RB = 2048
G = T // RB


IR = RB // 128


def _decode_body(g_ref, q_ref, gi_ref, q0_ref, q1_ref):
    iota_g = lax.broadcasted_iota(jnp.int32, (1, NG), 1).astype(jnp.float32)
    gi_ref[...] = jnp.sum(g_ref[...] * iota_g,
                          axis=1).astype(jnp.int32).reshape(IR, 128)
    iota_q = lax.broadcasted_iota(
        jnp.int32, (1, 1, NQ), 2).astype(jnp.float32)
    qidx = jnp.sum(q_ref[...] * iota_q, axis=2).astype(jnp.int32)
    q0_ref[...] = qidx[:, 0].reshape(IR, 128)
    q1_ref[...] = qidx[:, 1].reshape(IR, 128)


_decode = pl.pallas_call(
    _decode_body,
    grid=(G,),
    in_specs=[
        pl.BlockSpec((RB, NG), lambda i: (i, 0)),
        pl.BlockSpec((RB, 2, NQ), lambda i: (i, 0, 0)),
    ],
    out_specs=[
        pl.BlockSpec((IR, 128), lambda i: (i, 0)),
        pl.BlockSpec((IR, 128), lambda i: (i, 0)),
        pl.BlockSpec((IR, 128), lambda i: (i, 0)),
    ],
    out_shape=[jax.ShapeDtypeStruct((T // 128, 128), jnp.int32)] * 3,
)


_mesh = plsc.VectorSubcoreMesh(core_axis_name="c", subcore_axis_name="s")


@functools.partial(
    pl.kernel,
    out_type=jax.ShapeDtypeStruct((T, 2 * D), jnp.float32),
    mesh=_mesh,
    scratch_types=[
        pltpu.VMEM((NG, 2 * D), jnp.float32),
        pltpu.VMEM((NQ, D), jnp.float32),
        pltpu.VMEM((TPW,), jnp.int32),
        pltpu.VMEM((TPW,), jnp.int32),
        pltpu.VMEM((TPW,), jnp.int32),
        pltpu.VMEM((CH, 2 * D), jnp.float32),
        pltpu.VMEM((CH, 2 * D), jnp.float32),
        pltpu.SemaphoreType.DMA,
        pltpu.SemaphoreType.DMA,
    ],
)
def _sc_embed(gi_hbm, q0_hbm, q1_hbm, w_hbm, qt_hbm, out_hbm,
              w_v, qt_v, gi_v, q0i_v, q1i_v, ob0, ob1, osem0, osem1):
    wid = lax.axis_index("s") * 2 + lax.axis_index("c")
    base = wid * TPW
    pltpu.sync_copy(w_hbm, w_v)
    pltpu.sync_copy(qt_hbm, qt_v)
    pltpu.sync_copy(gi_hbm.at[pl.ds(base, TPW)], gi_v)
    pltpu.sync_copy(q0_hbm.at[pl.ds(base, TPW)], q0i_v)
    pltpu.sync_copy(q1_hbm.at[pl.ds(base, TPW)], q1i_v)

    bufs = (ob0, ob1)
    sems = (osem0, osem1)

    def pair(li, carry):
        for b in range(2):
            ci = 2 * li + b
            off = ci * CH
            @pl.when(li > 0)
            def _drain():
                prev = base + (ci - 2) * CH
                pltpu.make_async_copy(
                    bufs[b], out_hbm.at[pl.ds(prev, CH)], sems[b]).wait()

            def body_sg(sg, c2):
                s0 = sg * L
                gvec = gi_v[pl.ds(off + s0, L)]
                q0vec = q0i_v[pl.ds(off + s0, L)]
                q1vec = q1i_v[pl.ds(off + s0, L)]
                for k in range(L):
                    g = gvec[k]
                    q0 = q0vec[k]
                    q1 = q1vec[k]
                    s = s0 + k
                    for j in range(D // L):
                        bufs[b][s, pl.ds(j * L, L)] = (
                            w_v[g, pl.ds(j * L, L)]
                            + qt_v[q0, pl.ds(j * L, L)])
                        bufs[b][s, pl.ds(D + j * L, L)] = (
                            w_v[g, pl.ds(D + j * L, L)]
                            + qt_v[q1, pl.ds(j * L, L)])
                return c2

            lax.fori_loop(0, CH // L, body_sg, 0)
            pltpu.async_copy(
                bufs[b], out_hbm.at[pl.ds(base + off, CH)], sems[b])
        return carry

    lax.fori_loop(0, NCH // 2, pair, 0)
    for b in range(2):
        last = base + (NCH - 2 + b) * CH
        pltpu.make_async_copy(
            bufs[b], out_hbm.at[pl.ds(last, CH)], sems[b]).wait()


def kernel(gates_oh, gate_qubits_oh, qubits, W_gate):
    g_flat = gates_oh.reshape(T, NG)
    q_flat = gate_qubits_oh.reshape(T, 2, NQ)
    gi, q0i, q1i = _decode(g_flat, q_flat)
    out = _sc_embed(gi.reshape(T), q0i.reshape(T), q1i.reshape(T),
                    W_gate, qubits)
    return out.reshape(B, S, 2 * D)


# speedup vs baseline: 1.8094x; 1.2844x over previous
import functools

import jax
import jax.numpy as jnp
from jax import lax
from jax.experimental import pallas as pl
from jax.experimental.pallas import tpu as pltpu
from jax.experimental.pallas import tpu_sc as plsc

B, S = 4096, 50
NG = 32
NQ = 64
D = 128
T = B * S
NW = 32
TPW = T // NW
CH = 64
NCH = TPW // CH
L = 16


RB = 2048
G = T // RB
IR = RB // 128


def _decode_body(g_ref, q_ref, i0_ref, i1_ref):
    iota_g = lax.broadcasted_iota(jnp.int32, (1, NG), 1).astype(jnp.float32)
    gs = jnp.sum(g_ref[...] * iota_g, axis=1)
    iota_q = lax.broadcasted_iota(
        jnp.int32, (1, 1, NQ), 2).astype(jnp.float32)
    qs = jnp.sum(q_ref[...] * iota_q, axis=2)
    i0 = NQ * gs + qs[:, 0]
    i1 = (NG * NQ) + NQ * gs + qs[:, 1]
    i0_ref[...] = i0.astype(jnp.int32).reshape(IR, 128)
    i1_ref[...] = i1.astype(jnp.int32).reshape(IR, 128)


_decode = pl.pallas_call(
    _decode_body,
    grid=(G,),
    in_specs=[
        pl.BlockSpec((RB, NG), lambda i: (i, 0)),
        pl.BlockSpec((RB, 2, NQ), lambda i: (i, 0, 0)),
    ],
    out_specs=[
        pl.BlockSpec((IR, 128), lambda i: (i, 0)),
        pl.BlockSpec((IR, 128), lambda i: (i, 0)),
    ],
    out_shape=[jax.ShapeDtypeStruct((T // 128, 128), jnp.int32)] * 2,
)


def _table_body(w_ref, qt_ref, ct_ref):
    w = w_ref[...]
    qt = qt_ref[...]
    lo = w[:, None, :D] + qt[None, :, :]
    hi = w[:, None, D:] + qt[None, :, :]
    ct_ref[...] = jnp.concatenate(
        [lo.reshape(NG * NQ, D), hi.reshape(NG * NQ, D)], axis=0)


_build_table = pl.pallas_call(
    _table_body,
    out_shape=jax.ShapeDtypeStruct((2 * NG * NQ, D), jnp.float32),
)


_mesh = plsc.VectorSubcoreMesh(core_axis_name="c", subcore_axis_name="s")


@functools.partial(
    pl.kernel,
    out_type=jax.ShapeDtypeStruct((2 * T, D), jnp.float32),
    mesh=_mesh,
    compiler_params=pltpu.CompilerParams(needs_layout_passes=False),
    scratch_types=[
        pltpu.VMEM((TPW,), jnp.int32),
        pltpu.VMEM((TPW,), jnp.int32),
        pltpu.VMEM((2 * CH,), jnp.int32),
        pltpu.VMEM((2 * CH,), jnp.int32),
        pltpu.VMEM((2 * CH, D), jnp.float32),
        pltpu.VMEM((2 * CH, D), jnp.float32),
        pltpu.SemaphoreType.DMA,
        pltpu.SemaphoreType.DMA,
        pltpu.SemaphoreType.DMA,
        pltpu.SemaphoreType.DMA,
    ],
)
def _sc_gather(i0_hbm, i1_hbm, ct_hbm, out_hbm,
               i0_v, i1_v, ii0, ii1, ob0, ob1,
               gsem0, gsem1, osem0, osem1):
    wid = lax.axis_index("s") * 2 + lax.axis_index("c")
    base = wid * TPW
    pltpu.sync_copy(i0_hbm.at[pl.ds(base, TPW)], i0_v)
    pltpu.sync_copy(i1_hbm.at[pl.ds(base, TPW)], i1_v)

    iis = (ii0, ii1)
    obs = (ob0, ob1)
    gsems = (gsem0, gsem1)
    osems = (osem0, osem1)
    iota = lax.iota(jnp.int32, L)

    def start_gather(ci, b):
        off = ci * CH
        for m in range(CH // L):
            sl = pl.ds(off + m * L, L)
            plsc.store_scatter(iis[b], [2 * (m * L + iota)], i0_v[sl])
            plsc.store_scatter(iis[b], [2 * (m * L + iota) + 1], i1_v[sl])
        pltpu.async_copy(ct_hbm.at[iis[b]], obs[b], gsems[b])

    def finish_chunk(ci, b):
        pltpu.make_async_copy(ct_hbm.at[iis[b]], obs[b], gsems[b]).wait()
        dst = out_hbm.at[pl.ds(2 * (base + ci * CH), 2 * CH)]
        pltpu.async_copy(obs[b], dst, osems[b])

    def drain_out(ci, b):
        dst = out_hbm.at[pl.ds(2 * (base + ci * CH), 2 * CH)]
        pltpu.make_async_copy(obs[b], dst, osems[b]).wait()

    def pair(li, carry):
        for b in range(2):
            ci = 2 * li + b

            @pl.when(li > 0)
            def _reclaim():
                drain_out(ci - 2, b)

            start_gather(ci, b)
            if b == 0:

                @pl.when(li > 0)
                def _finish_prev():
                    finish_chunk(ci - 1, 1)

            else:
                finish_chunk(ci - 1, 0)
        return carry

    lax.fori_loop(0, NCH // 2, pair, 0)
    finish_chunk(NCH - 1, 1)
    drain_out(NCH - 2, 0)
    drain_out(NCH - 1, 1)


def kernel(gates_oh, gate_qubits_oh, qubits, W_gate):
    g_flat = gates_oh.reshape(T, NG)
    q_flat = gate_qubits_oh.reshape(T, 2, NQ)
    i0, i1 = _decode(g_flat, q_flat)
    ct = _build_table(W_gate, qubits)
    out = _sc_gather(i0.reshape(T), i1.reshape(T), ct)
    return out.reshape(B, S, 2 * D)
